# Initial kernel scaffold; baseline (speedup 1.0000x reference)
import jax
import jax.numpy as jnp
from jax.experimental import pallas as pl


def kernel(x, path_distance_map, edge_index_map, distance_embedding):
    raise NotImplementedError("write your pallas kernel here")


# speedup vs baseline: 24.0785x; 24.0785x over previous
import functools

import jax
import jax.numpy as jnp
from jax import lax
from jax.experimental import pallas as pl
from jax.experimental.pallas import tpu as pltpu
from jax.experimental.pallas import tpu_sc as plsc

B, T, Q = 8, 2048, 128
P = 1000000
MAX_PATH = 20

NC, NS = 2, 16
NW = NC * NS
ROWS_PER_W = T // NW
HALF_ROWS = 32

CHUNK = 4096
P_PAD = 1003520
NCHUNK = P_PAD // CHUNK
VEC_PER_CHUNK = CHUNK // 16

CAP = 48000


def _pack_body(n1_ref, n2_ref, d_ref, o_ref):
    o_ref[...] = (
        (n1_ref[...] << 16) | (n2_ref[...] << 5) | d_ref[...]
    )


def _sc_body(packed_hbm, emb_hbm, out_hbm, plist, buf, scanbuf, embv, sems,
             osem):
    wid = lax.axis_index("s") * NC + lax.axis_index("c")
    row0 = wid * ROWS_PER_W

    pltpu.sync_copy(emb_hbm, embv)
    cvec = plsc.load_gather(embv, [jnp.full((16,), MAX_PATH - 1, jnp.int32)])

    neg1 = jnp.full((16,), -1, jnp.int32)

    def _pf(i, _):
        plist[pl.ds(i * 16, 16)] = neg1
        return 0

    lax.fori_loop(0, CAP // 16, _pf, 0)

    def _start(c):
        slot = c & 1
        return pltpu.async_copy(
            packed_hbm.at[pl.ds(c * CHUNK, CHUNK)], scanbuf.at[slot],
            sems.at[slot])

    _start(0)

    def _chunk(c, cnt):
        slot = c & 1

        @pl.when(c + 1 < NCHUNK)
        def _():
            _start(c + 1)

        pltpu.make_async_copy(
            packed_hbm.at[pl.ds(c * CHUNK, CHUNK)], scanbuf.at[slot],
            sems.at[slot]).wait()

        def _vec(j, cnt):
            v = scanbuf[slot, pl.ds(j * 16, 16)]
            r = lax.shift_right_logical(v, 16) - row0
            m = (r >= 0) & (r < ROWS_PER_W)
            cnt_c = jnp.minimum(cnt, CAP - 16)
            plsc.store_compressed(plist.at[pl.ds(cnt_c, 16)], v, mask=m)
            pc = plsc.all_reduce_population_count(m)
            return cnt + pc[0]

        return lax.fori_loop(0, VEC_PER_CHUNK, _vec, cnt)

    cnt = lax.fori_loop(0, NCHUNK, _chunk, 0)
    nvec = (cnt + 15) // 16

    for h in range(2):
        rowbase = row0 + h * HALF_ROWS

        def _init_row(i, _):
            def _init_v(j, _):
                buf[i, pl.ds(j * 16, 16)] = cvec
                return 0
            lax.fori_loop(0, T // 16, _init_v, 0)
            return 0

        lax.fori_loop(0, HALF_ROWS, _init_row, 0)

        def _scat(j, _):
            v = plist[pl.ds(j * 16, 16)]
            rh = lax.shift_right_logical(v, 16) - rowbase
            m = (rh >= 0) & (rh < HALF_ROWS)
            n2 = lax.shift_right_logical(v, 5) & 0x7FF
            d = v & 0x1F
            val = plsc.load_gather(embv, [d])
            plsc.store_scatter(buf, [rh, n2], val, mask=m)
            return 0

        lax.fori_loop(0, nvec, _scat, 0)

        for b in range(B):
            pltpu.async_copy(
                buf, out_hbm.at[b, pl.ds(rowbase, HALF_ROWS), :], osem)
        for b in range(B):
            pltpu.make_async_copy(
                buf, out_hbm.at[b, pl.ds(rowbase, HALF_ROWS), :], osem).wait()


@jax.jit
def kernel(x, path_distance_map, edge_index_map, distance_embedding):
    del x
    n1 = edge_index_map[:, 0]
    n2 = edge_index_map[:, 1]
    d = path_distance_map
    pad = P_PAD - P
    n1 = jnp.concatenate([n1, jnp.full((pad,), T, jnp.int32)]).reshape(-1, 128)
    n2 = jnp.concatenate([n2, jnp.zeros((pad,), jnp.int32)]).reshape(-1, 128)
    d = jnp.concatenate([d, jnp.zeros((pad,), jnp.int32)]).reshape(-1, 128)

    rows = P_PAD // 128
    blk = rows // 10
    packed = pl.pallas_call(
        _pack_body,
        out_shape=jax.ShapeDtypeStruct((rows, 128), jnp.int32),
        grid=(10,),
        in_specs=[pl.BlockSpec((blk, 128), lambda i: (i, 0))] * 3,
        out_specs=pl.BlockSpec((blk, 128), lambda i: (i, 0)),
    )(n1, n2, d).reshape(P_PAD)

    emb32 = jnp.concatenate(
        [distance_embedding[:, 0],
         jnp.zeros((32 - MAX_PATH,), jnp.float32)])

    sc = pl.kernel(
        _sc_body,
        out_type=jax.ShapeDtypeStruct((B, T, T), jnp.float32),
        mesh=plsc.VectorSubcoreMesh(core_axis_name="c", subcore_axis_name="s"),
        scratch_types=[
            pltpu.VMEM((CAP,), jnp.int32),
            pltpu.VMEM((HALF_ROWS, T), jnp.float32),
            pltpu.VMEM((2, CHUNK), jnp.int32),
            pltpu.VMEM((32,), jnp.float32),
            pltpu.SemaphoreType.DMA((2,)),
            pltpu.SemaphoreType.DMA,
        ],
        compiler_params=pltpu.CompilerParams(needs_layout_passes=False),
    )
    return sc(packed, emb32)


# speedup vs baseline: 25.1013x; 1.0425x over previous
import functools

import jax
import jax.numpy as jnp
from jax import lax
from jax.experimental import pallas as pl
from jax.experimental.pallas import tpu as pltpu
from jax.experimental.pallas import tpu_sc as plsc

B, T, Q = 8, 2048, 128
P = 1000000
MAX_PATH = 20

NC, NS = 2, 16
NW = NC * NS
ROWS_PER_W = T // NW
HALF_ROWS = 32

NSEG = 1
CHUNK = 2048
P_PAD = 1015808
NCHUNK = P_PAD // CHUNK
SEG_CHUNKS = NCHUNK // NSEG
VEC_PER_CHUNK = CHUNK // 16

CAP_SEG = 48000


def _pack_body(n1_ref, n2_ref, d_ref, o_ref):
    o_ref[...] = (
        (n1_ref[...] << 16) | (n2_ref[...] << 5) | d_ref[...]
    )


def _bcast_body(p_ref, o_ref):
    o_ref[...] = jnp.broadcast_to(p_ref[...][None], o_ref.shape)


def _sc_body(packed_hbm, emb_hbm, plane_hbm, plist, buf, scanbuf, embv, sems,
             osem):
    wid = lax.axis_index("s") * NC + lax.axis_index("c")
    row0 = wid * ROWS_PER_W

    pltpu.sync_copy(emb_hbm, embv)
    cvec = plsc.load_gather(embv, [jnp.full((16,), MAX_PATH - 1, jnp.int32)])

    neg1 = jnp.full((16,), -1, jnp.int32)

    def _pf(i, _):
        for q in range(NSEG):
            plist[q, pl.ds(i * 16, 16)] = neg1
        return 0

    lax.fori_loop(0, CAP_SEG // 16, _pf, 0)

    def _start(q, c):
        slot = 2 * q + (c & 1)
        return pltpu.async_copy(
            packed_hbm.at[pl.ds((q * SEG_CHUNKS + c) * CHUNK, CHUNK)],
            scanbuf.at[slot], sems.at[slot])

    for q in range(NSEG):
        _start(q, 0)

    def _chunk(c, cnts):
        for q in range(NSEG):
            @pl.when(c + 1 < SEG_CHUNKS)
            def _():
                _start(q, c + 1)
        for q in range(NSEG):
            slot = 2 * q + (c & 1)
            pltpu.make_async_copy(
                packed_hbm.at[pl.ds((q * SEG_CHUNKS + c) * CHUNK, CHUNK)],
                scanbuf.at[slot], sems.at[slot]).wait()

        def _vec(j, cnts):
            out = []
            for q in range(NSEG):
                slot = 2 * q + (c & 1)
                v = scanbuf[slot, pl.ds(j * 16, 16)]
                r = lax.shift_right_logical(v, 16) - row0
                m = (r >= 0) & (r < ROWS_PER_W)
                cnt_c = jnp.minimum(cnts[q], CAP_SEG - 16)
                plsc.store_compressed(
                    plist.at[q, pl.ds(cnt_c, 16)], v, mask=m)
                pc = plsc.all_reduce_population_count(m)
                out.append(cnts[q] + pc[0])
            return tuple(out)

        return lax.fori_loop(0, VEC_PER_CHUNK, _vec, cnts)

    cnts = lax.fori_loop(0, SEG_CHUNKS, _chunk, (0,) * NSEG)
    nvecs = [(cnts[q] + 15) // 16 for q in range(NSEG)]

    for h in range(2):
        rowbase = row0 + h * HALF_ROWS

        def _init_row(i, _):
            def _init_v(j, _):
                for u in range(8):
                    buf[i, pl.ds(j * 128 + u * 16, 16)] = cvec
                return 0
            lax.fori_loop(0, T // 128, _init_v, 0)
            return 0

        lax.fori_loop(0, HALF_ROWS, _init_row, 0)

        for q in range(NSEG):
            def _scat(j, _):
                v = plist[q, pl.ds(j * 16, 16)]
                rh = lax.shift_right_logical(v, 16) - rowbase
                m = (rh >= 0) & (rh < HALF_ROWS)
                n2 = lax.shift_right_logical(v, 5) & 0x7FF
                d = v & 0x1F
                val = plsc.load_gather(embv, [d])
                plsc.store_scatter(buf, [rh, n2], val, mask=m)
                return 0

            lax.fori_loop(0, nvecs[q], _scat, 0)

        pltpu.async_copy(
            buf, plane_hbm.at[pl.ds(rowbase, HALF_ROWS), :], osem)
        pltpu.make_async_copy(
            buf, plane_hbm.at[pl.ds(rowbase, HALF_ROWS), :], osem).wait()


@jax.jit
def kernel(x, path_distance_map, edge_index_map, distance_embedding):
    del x
    n1 = edge_index_map[:, 0]
    n2 = edge_index_map[:, 1]
    d = path_distance_map
    pad = P_PAD - P
    n1 = jnp.concatenate([n1, jnp.full((pad,), T, jnp.int32)]).reshape(-1, 128)
    n2 = jnp.concatenate([n2, jnp.zeros((pad,), jnp.int32)]).reshape(-1, 128)
    d = jnp.concatenate([d, jnp.zeros((pad,), jnp.int32)]).reshape(-1, 128)

    rows = P_PAD // 128
    blk = rows // 8
    packed = pl.pallas_call(
        _pack_body,
        out_shape=jax.ShapeDtypeStruct((rows, 128), jnp.int32),
        grid=(8,),
        in_specs=[pl.BlockSpec((blk, 128), lambda i: (i, 0))] * 3,
        out_specs=pl.BlockSpec((blk, 128), lambda i: (i, 0)),
    )(n1, n2, d).reshape(P_PAD)

    emb32 = jnp.concatenate(
        [distance_embedding[:, 0],
         jnp.zeros((32 - MAX_PATH,), jnp.float32)])

    sc = pl.kernel(
        _sc_body,
        out_type=jax.ShapeDtypeStruct((T, T), jnp.float32),
        mesh=plsc.VectorSubcoreMesh(core_axis_name="c", subcore_axis_name="s"),
        scratch_types=[
            pltpu.VMEM((NSEG, CAP_SEG), jnp.int32),
            pltpu.VMEM((HALF_ROWS, T), jnp.float32),
            pltpu.VMEM((2 * NSEG, CHUNK), jnp.int32),
            pltpu.VMEM((32,), jnp.float32),
            pltpu.SemaphoreType.DMA((2 * NSEG,)),
            pltpu.SemaphoreType.DMA,
        ],
        compiler_params=pltpu.CompilerParams(needs_layout_passes=False),
    )
    plane = sc(packed, emb32)

    RB = 256
    return pl.pallas_call(
        _bcast_body,
        out_shape=jax.ShapeDtypeStruct((B, T, T), jnp.float32),
        grid=(T // RB,),
        in_specs=[pl.BlockSpec((RB, T), lambda i: (i, 0))],
        out_specs=pl.BlockSpec((B, RB, T), lambda i: (0, i, 0)),
    )(plane)
